# baseline (speedup 1.0000x reference)
import functools

import jax
import jax.numpy as jnp
from jax import lax
from jax.experimental import pallas as pl
from jax.experimental.pallas import tpu as pltpu
from jax.experimental.pallas import tpu_sc as plsc

_OOV = 100
_LANES = 16
_TILE = 128
_SUB = 8
_EB = 16
_SEG = 2048


def _final_dist_sc_lin(B, V, L, NPAD):
    E = V + _OOV
    ET = (E // _EB) * _EB
    TAILR = E - ET
    NBT = ET // _EB
    NVB = V // _EB
    NW = 32
    GMAX = -(-NBT // NW)
    NOFF = ((NBT + 2 + 15) // 16 + 1) * 16
    NTPB = _EB // _SUB * (B // _TILE)
    BW = _EB * B

    info = plsc.get_sparse_core_info()
    assert info.num_cores * info.num_subcores == NW

    mesh = plsc.VectorSubcoreMesh(core_axis_name="c", subcore_axis_name="s")

    def iota16():
        return lax.iota(jnp.int32, _LANES)

    @functools.partial(
        pl.kernel,
        mesh=mesh,
        compiler_params=pltpu.CompilerParams(needs_layout_passes=False),
        out_type=jax.ShapeDtypeStruct((E * B,), jnp.float32),
        scratch_types=[
            pltpu.VMEM((NTPB, _SUB, _TILE), jnp.float32),
            pltpu.VMEM((NTPB, _SUB, _TILE), jnp.float32),
            pltpu.VMEM((NTPB, _SUB, _TILE), jnp.float32),
            pltpu.VMEM((BW,), jnp.float32),
            pltpu.VMEM((BW,), jnp.float32),
            pltpu.VMEM((BW,), jnp.float32),
            pltpu.VMEM((TAILR * B,), jnp.float32),
            pltpu.VMEM((NOFF,), jnp.int32),
            pltpu.VMEM((B,), jnp.float32),
            pltpu.VMEM((_SEG,), jnp.int32),
            pltpu.VMEM((_SEG,), jnp.int32),
            pltpu.VMEM((_SEG,), jnp.float32),
            pltpu.SemaphoreType.DMA,
            pltpu.SemaphoreType.DMA,
            pltpu.SemaphoreType.DMA,
            pltpu.SemaphoreType.DMA,
            pltpu.SemaphoreType.DMA,
            pltpu.SemaphoreType.DMA,
        ],
    )
    def sc_fn(vocab_hbm, ids_hbm, bs_hbm, val_hbm, off_hbm, pg_hbm,
              out_hbm, st0, st1, st2, lb0, lb1, lb2, tailb, offv, pgb,
              idsv, bsv, valv, si0, si1, si2, so0, so1, so2):
        wid = lax.axis_index("s") * info.num_cores + lax.axis_index("c")
        sts = (st0, st1, st2)
        lbs = (lb0, lb1, lb2)
        isems = (si0, si1, si2)
        osems = (so0, so1, so2)

        pltpu.sync_copy(off_hbm, offv)
        pltpu.sync_copy(pg_hbm, pgb)

        def blkid_of(g):
            return wid + NW * g

        def off_at(i):
            o16 = offv[pl.ds(i, _LANES)]
            return jnp.sum(jnp.where(iota16() == 0, o16, 0))

        def fire_in(st, sem, g):
            blkid = blkid_of(g)
            e0 = blkid * _EB

            @pl.when(blkid < NVB)
            def _():
                for j in range(NTPB):
                    s, bt = divmod(j, B // _TILE)
                    pltpu.make_async_copy(
                        vocab_hbm.at[pl.ds(e0 + _SUB * s, _SUB),
                                     pl.ds(_TILE * bt, _TILE)],
                        st.at[j], sem).start()

        def wait_in(st, sem, g):
            blkid = blkid_of(g)
            e0 = blkid * _EB

            @pl.when(blkid < NVB)
            def _():
                for j in range(NTPB):
                    s, bt = divmod(j, B // _TILE)
                    pltpu.make_async_copy(
                        vocab_hbm.at[pl.ds(e0 + _SUB * s, _SUB),
                                     pl.ds(_TILE * bt, _TILE)],
                        st.at[j], sem).wait()

        def fire_out(lb, sem, g):
            blkid = blkid_of(g)

            @pl.when(blkid < NBT)
            def _():
                pltpu.make_async_copy(
                    lb, out_hbm.at[pl.ds(blkid * BW, BW)], sem).start()

        def wait_out(lb, sem, g):
            blkid = blkid_of(g)

            @pl.when(blkid < NBT)
            def _():
                pltpu.make_async_copy(
                    lb, out_hbm.at[pl.ds(blkid * BW, BW)], sem).wait()

        def seg_scatter(dstref, s0, s1, ebase, erange):
            a0 = (s0 // 8) * 8
            nk = (s1 - a0 + _SEG - 1) // _SEG

            def kbody(k, c):
                base = a0 + k * _SEG
                pltpu.sync_copy(ids_hbm.at[pl.ds(base, _SEG)], idsv)
                pltpu.sync_copy(bs_hbm.at[pl.ds(base, _SEG)], bsv)
                pltpu.sync_copy(val_hbm.at[pl.ds(base, _SEG)], valv)
                nch = jnp.minimum(
                    _SEG // _LANES,
                    (s1 - base + _LANES - 1) // _LANES)

                def cbody(c2, c3):
                    sl = pl.ds(c2 * _LANES, _LANES)
                    gidx = base + c2 * _LANES + iota16()
                    m = (gidx >= s0) & (gidx < s1)
                    el = idsv[sl] - ebase
                    m = m & (el >= 0) & (el < erange)
                    plsc.addupdate_scatter(
                        dstref, [el * B + bsv[sl]], valv[sl], mask=m)
                    return c3
                lax.fori_loop(0, nch, cbody, None)
                return c
            lax.fori_loop(0, nk, kbody, None)

        def process(st, lb, g):
            blkid = blkid_of(g)

            @pl.when(blkid >= NVB)
            def _():
                zero = jnp.zeros((_LANES,), jnp.float32)

                def zbody(z, c):
                    for i in range(16):
                        lb[pl.ds((z * 16 + i) * _LANES, _LANES)] = zero
                    return c
                lax.fori_loop(0, BW // _LANES // 16, zbody, None)

            @pl.when(blkid < NVB)
            def _():
                def dbody(j, c):
                    s = lax.shift_right_logical(j, 3)
                    bt = lax.bitwise_and(j, B // _TILE - 1)
                    pgc = [pgb[pl.ds(bt * _TILE + c2 * _LANES, _LANES)]
                           for c2 in range(_TILE // _LANES)]
                    for sub in range(_SUB):
                        ebase = (s * _SUB + sub) * B + bt * _TILE
                        for c2 in range(_TILE // _LANES):
                            lb[pl.ds(ebase + c2 * _LANES, _LANES)] = (
                                st[j, sub, pl.ds(c2 * _LANES, _LANES)]
                                * pgc[c2])
                    return c
                lax.fori_loop(0, NTPB, dbody, None)

            @pl.when(blkid < NBT)
            def _():
                s0 = off_at(blkid)
                s1 = off_at(blkid + 1)
                seg_scatter(lb, s0, s1, blkid * _EB, _EB)

        fire_in(st0, si0, 0)
        fire_in(st1, si1, 1)

        def blk_step(g, c2):
            for b in range(3):
                @pl.when(g % 3 == b)
                def _():
                    wait_in(sts[b], isems[b], g)
                    process(sts[b], lbs[b], g)
                    fire_out(lbs[b], osems[b], g)
                    bp = (b + 2) % 3

                    @pl.when(g + 2 < GMAX)
                    def _():
                        @pl.when(g >= 1)
                        def _():
                            wait_out(lbs[bp], osems[bp], g - 1)
                        fire_in(sts[bp], isems[bp], g + 2)
            return c2

        lax.fori_loop(0, GMAX, blk_step, None)
        for g in (GMAX - 3, GMAX - 2, GMAX - 1):
            wait_out(lbs[g % 3], osems[g % 3], g)

        @pl.when(wid == NW - 1)
        def _():
            zero = jnp.zeros((_LANES,), jnp.float32)

            def ztail(z, c):
                for i in range(16):
                    tailb[pl.ds((z * 16 + i) * _LANES, _LANES)] = zero
                return c
            lax.fori_loop(0, TAILR * B // _LANES // 16, ztail, None)
            s0 = off_at(NBT)
            s1 = off_at(NBT + 1)
            seg_scatter(tailb, s0, s1, ET, TAILR)
            pltpu.sync_copy(tailb, out_hbm.at[pl.ds(ET * B, TAILR * B)])

    return sc_fn, NBT, NOFF


def kernel(vocab_dists, attn_dists, p_gens, input_ids):
    T, B, V = vocab_dists.shape
    L = attn_dists.shape[-1]
    E = V + _OOV
    N = B * L
    NPAD = N + _SEG + 8

    sc_fn, NBT, NOFF = _final_dist_sc_lin(B, V, L, NPAD)

    outs = []
    for t in range(T):
        vocab_T = jnp.swapaxes(vocab_dists[t], 0, 1)
        pg_b = p_gens[t, :, 0]

        idsf = input_ids.reshape(-1)
        iota = jnp.arange(N, dtype=jnp.int32)
        ids_s, perm = lax.sort([idsf, iota], num_keys=1)
        valf = ((1.0 - pg_b)[:, None] * attn_dists[t]).reshape(-1)
        val_s = valf[perm]
        b_s = (perm // L).astype(jnp.int32)

        sentinel = jnp.int32(0x3FFFFFFF)
        ids_p = jnp.concatenate(
            [ids_s, jnp.full((NPAD - N,), sentinel, jnp.int32)])
        bs_p = jnp.concatenate([b_s, jnp.zeros((NPAD - N,), jnp.int32)])
        val_p = jnp.concatenate([val_s, jnp.zeros((NPAD - N,), jnp.float32)])

        edges = jnp.arange(NBT + 2, dtype=jnp.int32) * _EB
        offs = jnp.searchsorted(ids_s, edges).astype(jnp.int32)
        offs_p = jnp.concatenate(
            [offs, jnp.full((NOFF - NBT - 2,), N, jnp.int32)])

        out_lin = sc_fn(vocab_T, ids_p, bs_p, val_p, offs_p, pg_b)
        out_T = out_lin.reshape(E, B)
        outs.append(jnp.swapaxes(out_T, 0, 1))
    return jnp.stack(outs, axis=0)


# speedup vs baseline: 1.4115x; 1.4115x over previous
import functools

import jax
import jax.numpy as jnp
from jax import lax
from jax.experimental import pallas as pl
from jax.experimental.pallas import tpu as pltpu
from jax.experimental.pallas import tpu_sc as plsc

_OOV = 100
_LANES = 16
_TILE = 128
_SUB = 8
_EB = 16
_SEG = 2048


def _final_dist_sc_lin(B, V, L, NPAD):
    E = V + _OOV
    ET = (E // _EB) * _EB
    TAILR = E - ET
    NBT = ET // _EB
    NVB = V // _EB
    NW = 32
    GMAX = -(-NBT // NW)
    NOFF = ((NBT + 2 + 15) // 16 + 1) * 16
    NTPB = _EB // _SUB * (B // _TILE)
    BW = _EB * B

    info = plsc.get_sparse_core_info()
    assert info.num_cores * info.num_subcores == NW

    mesh = plsc.VectorSubcoreMesh(core_axis_name="c", subcore_axis_name="s")

    def iota16():
        return lax.iota(jnp.int32, _LANES)

    @functools.partial(
        pl.kernel,
        mesh=mesh,
        compiler_params=pltpu.CompilerParams(needs_layout_passes=False),
        out_type=jax.ShapeDtypeStruct((E * B,), jnp.float32),
        scratch_types=[
            pltpu.VMEM((NTPB, _SUB, _TILE), jnp.float32),
            pltpu.VMEM((NTPB, _SUB, _TILE), jnp.float32),
            pltpu.VMEM((NTPB, _SUB, _TILE), jnp.float32),
            pltpu.VMEM((BW,), jnp.float32),
            pltpu.VMEM((BW,), jnp.float32),
            pltpu.VMEM((BW,), jnp.float32),
            pltpu.VMEM((TAILR * B,), jnp.float32),
            pltpu.VMEM((NOFF,), jnp.int32),
            pltpu.VMEM((B,), jnp.float32),
            pltpu.VMEM((_SEG,), jnp.int32),
            pltpu.VMEM((_SEG,), jnp.int32),
            pltpu.VMEM((_SEG,), jnp.int32),
            pltpu.SemaphoreType.DMA,
            pltpu.SemaphoreType.DMA,
            pltpu.SemaphoreType.DMA,
            pltpu.SemaphoreType.DMA,
            pltpu.SemaphoreType.DMA,
            pltpu.SemaphoreType.DMA,
            pltpu.SemaphoreType.DMA,
            pltpu.SemaphoreType.DMA,
            pltpu.SemaphoreType.DMA,
        ],
    )
    def sc_fn(vocab_hbm, quad_hbm, off_hbm, pg_hbm,
              out_hbm, st0, st1, st2, lb0, lb1, lb2, tailb, offv, pgb,
              qb0, qb1, qb2, si0, si1, si2, so0, so1, so2, sq0, sq1, sq2):
        wid = lax.axis_index("s") * info.num_cores + lax.axis_index("c")
        sts = (st0, st1, st2)
        lbs = (lb0, lb1, lb2)
        qbs = (qb0, qb1, qb2)
        isems = (si0, si1, si2)
        osems = (so0, so1, so2)
        qsems = (sq0, sq1, sq2)

        pltpu.sync_copy(off_hbm, offv)
        pltpu.sync_copy(pg_hbm, pgb)

        def blkid_of(g):
            return wid + NW * g

        def off_at(i):
            o16 = offv[pl.ds(i, _LANES)]
            return jnp.sum(jnp.where(iota16() == 0, o16, 0))

        def fire_in(st, sem, g):
            blkid = blkid_of(g)
            e0 = blkid * _EB

            @pl.when(blkid < NVB)
            def _():
                for j in range(NTPB):
                    s, bt = divmod(j, B // _TILE)
                    pltpu.make_async_copy(
                        vocab_hbm.at[pl.ds(e0 + _SUB * s, _SUB),
                                     pl.ds(_TILE * bt, _TILE)],
                        st.at[j], sem).start()

        def wait_in(st, sem, g):
            blkid = blkid_of(g)
            e0 = blkid * _EB

            @pl.when(blkid < NVB)
            def _():
                for j in range(NTPB):
                    s, bt = divmod(j, B // _TILE)
                    pltpu.make_async_copy(
                        vocab_hbm.at[pl.ds(e0 + _SUB * s, _SUB),
                                     pl.ds(_TILE * bt, _TILE)],
                        st.at[j], sem).wait()

        def fire_out(lb, sem, g):
            blkid = blkid_of(g)

            @pl.when(blkid < NBT)
            def _():
                pltpu.make_async_copy(
                    lb, out_hbm.at[pl.ds(blkid * BW, BW)], sem).start()

        def wait_out(lb, sem, g):
            blkid = blkid_of(g)

            @pl.when(blkid < NBT)
            def _():
                pltpu.make_async_copy(
                    lb, out_hbm.at[pl.ds(blkid * BW, BW)], sem).wait()

        TRI = _SEG // 4

        def fire_seg(qb, qs, g):
            blkid = blkid_of(g)

            @pl.when(blkid < NBT)
            def _():
                s0 = off_at(blkid)
                a0 = (s0 // 8) * 8
                pltpu.make_async_copy(
                    quad_hbm.at[pl.ds(a0 * 4, _SEG)], qb, qs).start()

        def seg_walk(dstref, qb, qs, s0, s1, ebase, erange, prefetched):
            a0 = (s0 // 8) * 8
            nk = (s1 - a0 + TRI - 1) // TRI
            if prefetched:
                nk = jnp.maximum(nk, 1)

            def kbody(k, c):
                base = a0 + k * TRI
                src = quad_hbm.at[pl.ds(base * 4, _SEG)]
                if prefetched:
                    @pl.when(k > 0)
                    def _():
                        pltpu.make_async_copy(src, qb, qs).start()
                else:
                    pltpu.make_async_copy(src, qb, qs).start()
                pltpu.make_async_copy(src, qb, qs).wait()
                nch = jnp.minimum(
                    TRI // _LANES,
                    (s1 - base + _LANES - 1) // _LANES)

                def cbody(c2, c3):
                    gidx = base + c2 * _LANES + iota16()
                    m = (gidx >= s0) & (gidx < s1)
                    qidx = (c2 * _LANES + iota16()) * 4
                    idc = plsc.load_gather(qb, [qidx])
                    bc = plsc.load_gather(qb, [qidx + 1])
                    vc = plsc.bitcast(
                        plsc.load_gather(qb, [qidx + 2]), jnp.float32)
                    el = idc - ebase
                    m = m & (el >= 0) & (el < erange)
                    plsc.addupdate_scatter(
                        dstref, [el * B + bc], vc, mask=m)
                    return c3
                lax.fori_loop(0, nch, cbody, None)
                return c
            lax.fori_loop(0, nk, kbody, None)

        def process(st, lb, qb, qs, g):
            blkid = blkid_of(g)

            @pl.when(blkid >= NVB)
            def _():
                zero = jnp.zeros((_LANES,), jnp.float32)

                def zbody(z, c):
                    for i in range(16):
                        lb[pl.ds((z * 16 + i) * _LANES, _LANES)] = zero
                    return c
                lax.fori_loop(0, BW // _LANES // 16, zbody, None)

            @pl.when(blkid < NVB)
            def _():
                def dbody(j, c):
                    s = lax.shift_right_logical(j, 3)
                    bt = lax.bitwise_and(j, B // _TILE - 1)
                    pgc = [pgb[pl.ds(bt * _TILE + c2 * _LANES, _LANES)]
                           for c2 in range(_TILE // _LANES)]
                    for sub in range(_SUB):
                        ebase = (s * _SUB + sub) * B + bt * _TILE
                        for c2 in range(_TILE // _LANES):
                            lb[pl.ds(ebase + c2 * _LANES, _LANES)] = (
                                st[j, sub, pl.ds(c2 * _LANES, _LANES)]
                                * pgc[c2])
                    return c
                lax.fori_loop(0, NTPB, dbody, None)

            @pl.when(blkid < NBT)
            def _():
                s0 = off_at(blkid)
                s1 = off_at(blkid + 1)
                seg_walk(lb, qb, qs, s0, s1, blkid * _EB, _EB, True)

        fire_in(st0, si0, 0)
        fire_in(st1, si1, 1)
        fire_seg(qb0, sq0, 0)
        fire_seg(qb1, sq1, 1)

        def blk_step(g, c2):
            for b in range(3):
                @pl.when(g % 3 == b)
                def _():
                    wait_in(sts[b], isems[b], g)
                    process(sts[b], lbs[b], qbs[b], qsems[b], g)
                    fire_out(lbs[b], osems[b], g)
                    bp = (b + 2) % 3

                    @pl.when(g + 2 < GMAX)
                    def _():
                        @pl.when(g >= 1)
                        def _():
                            wait_out(lbs[bp], osems[bp], g - 1)
                        fire_in(sts[bp], isems[bp], g + 2)
                        fire_seg(qbs[bp], qsems[bp], g + 2)
            return c2

        lax.fori_loop(0, GMAX, blk_step, None)
        for g in (GMAX - 3, GMAX - 2, GMAX - 1):
            wait_out(lbs[g % 3], osems[g % 3], g)

        @pl.when(wid == NW - 1)
        def _():
            zero = jnp.zeros((_LANES,), jnp.float32)

            def ztail(z, c):
                for i in range(16):
                    tailb[pl.ds((z * 16 + i) * _LANES, _LANES)] = zero
                return c
            lax.fori_loop(0, TAILR * B // _LANES // 16, ztail, None)
            s0 = off_at(NBT)
            s1 = off_at(NBT + 1)
            seg_walk(tailb, qb0, sq0, s0, s1, ET, TAILR, False)
            pltpu.sync_copy(tailb, out_hbm.at[pl.ds(ET * B, TAILR * B)])

    return sc_fn, NBT, NOFF


def kernel(vocab_dists, attn_dists, p_gens, input_ids):
    T, B, V = vocab_dists.shape
    L = attn_dists.shape[-1]
    E = V + _OOV
    N = B * L
    NPAD = N + _SEG // 4 + 8

    sc_fn, NBT, NOFF = _final_dist_sc_lin(B, V, L, NPAD)

    outs = []
    for t in range(T):
        vocab_T = jnp.swapaxes(vocab_dists[t], 0, 1)
        pg_b = p_gens[t, :, 0]

        idsf = input_ids.reshape(-1)
        iota = jnp.arange(N, dtype=jnp.int32)
        ids_s, perm = lax.sort([idsf, iota], num_keys=1)
        valf = ((1.0 - pg_b)[:, None] * attn_dists[t]).reshape(-1)
        val_s = valf[perm]
        b_s = (perm // L).astype(jnp.int32)

        sentinel = jnp.int32(0x3FFFFFFF)
        npd = NPAD - N
        ids_p = jnp.concatenate(
            [ids_s, jnp.full((npd,), sentinel, jnp.int32)])
        bs_p = jnp.concatenate([b_s, jnp.zeros((npd,), jnp.int32)])
        vb_p = jnp.concatenate(
            [lax.bitcast_convert_type(val_s, jnp.int32),
             jnp.zeros((npd,), jnp.int32)])
        quad = jnp.stack(
            [ids_p, bs_p, vb_p, jnp.zeros((NPAD,), jnp.int32)],
            axis=1).reshape(-1)

        counts = jnp.zeros((NBT + 2,), jnp.int32).at[idsf // _EB].add(1)
        offs = jnp.concatenate(
            [jnp.zeros((1,), jnp.int32),
             jnp.cumsum(counts, dtype=jnp.int32)])[: NBT + 2]
        offs_p = jnp.concatenate(
            [offs, jnp.full((NOFF - NBT - 2,), N, jnp.int32)])

        out_lin = sc_fn(vocab_T, quad, offs_p, pg_b)
        out_T = out_lin.reshape(E, B)
        outs.append(jnp.swapaxes(out_T, 0, 1))
    return jnp.stack(outs, axis=0)


# speedup vs baseline: 3.0387x; 2.1528x over previous
import functools

import jax
import jax.numpy as jnp
from jax import lax
from jax.experimental import pallas as pl
from jax.experimental.pallas import tpu as pltpu
from jax.experimental.pallas import tpu_sc as plsc

_OOV = 100
_LANES = 16
_TILE = 128
_SUB = 8


def _final_dist_sc_tiled(B, V, L):
    E = V + _OOV
    ET = (E // _TILE) * _TILE
    TAIL = E - ET
    TAILP = _LANES
    NT = ET // _TILE
    NVT = V // _TILE
    TB = 37
    NBLK = -(-NT // TB)
    LP = ((L + _LANES - 1) // _LANES) * _LANES
    VTW = V - NVT * _TILE

    info = plsc.get_sparse_core_info()
    NW = info.num_cores * info.num_subcores
    NS = B // _SUB
    SPW = NS // NW

    mesh = plsc.VectorSubcoreMesh(core_axis_name="c", subcore_axis_name="s")

    @functools.partial(
        pl.kernel,
        mesh=mesh,
        compiler_params=pltpu.CompilerParams(needs_layout_passes=False),
        out_type=(
            jax.ShapeDtypeStruct((B, E), jnp.float32),
            jax.ShapeDtypeStruct((B * TAILP,), jnp.float32),
        ),
        scratch_types=[
            pltpu.VMEM((TB, _SUB, _TILE), jnp.float32),
            pltpu.VMEM((TB, _SUB, _TILE), jnp.float32),
            pltpu.VMEM((TB, _SUB, _TILE), jnp.float32),
            pltpu.VMEM((_SUB * TAILP,), jnp.float32),
            pltpu.VMEM((_SUB * LP,), jnp.float32),
            pltpu.VMEM((_SUB * LP,), jnp.int32),
            pltpu.VMEM((_SUB * _LANES,), jnp.float32),
            pltpu.VMEM((_SUB * VTW,), jnp.float32),
            pltpu.SemaphoreType.DMA,
            pltpu.SemaphoreType.DMA,
            pltpu.SemaphoreType.DMA,
            pltpu.SemaphoreType.DMA,
            pltpu.SemaphoreType.DMA,
            pltpu.SemaphoreType.DMA,
        ],
    )
    def sc_fn(vocab_hbm, attn_hbm, ids_hbm, pg_hbm, vt_hbm,
              out_hbm, tail_hbm, buf0, buf1, buf2, tailb, attnb, idsb,
              pgb, vtb, si0, si1, si2, so0, so1, so2):
        wid = lax.axis_index("s") * info.num_cores + lax.axis_index("c")
        bufs = (buf0, buf1, buf2)
        isems = (si0, si1, si2)
        osems = (so0, so1, so2)

        def nin_of(blk):
            blk = jnp.asarray(blk, jnp.int32)
            tb = jnp.minimum(TB, NT - blk * TB)
            return tb - (blk == NBLK - 1).astype(jnp.int32)

        def stripe_body(si, carry):
            s = wid * SPW + si
            r0 = s * _SUB
            pltpu.sync_copy(attn_hbm.at[pl.ds(r0 * LP, _SUB * LP)], attnb)
            pltpu.sync_copy(ids_hbm.at[pl.ds(r0 * LP, _SUB * LP)], idsb)
            pltpu.sync_copy(pg_hbm.at[pl.ds(r0 * _LANES, _SUB * _LANES)], pgb)
            pltpu.sync_copy(vt_hbm.at[pl.ds(r0 * VTW, _SUB * VTW)], vtb)

            pgs = [pgb[pl.ds(r * _LANES, _LANES)] for r in range(_SUB)]

            for r in range(_SUB):
                omp = 1.0 - pgs[r]
                for j in range(LP // _LANES):
                    sl = pl.ds(r * LP + j * _LANES, _LANES)
                    attnb[sl] = attnb[sl] * omp

            for i in range(_SUB * TAILP // _LANES):
                tailb[pl.ds(i * _LANES, _LANES)] = jnp.zeros(
                    (_LANES,), jnp.float32)
            for r in range(_SUB):
                for j in range(LP // _LANES):
                    sl = pl.ds(r * LP + j * _LANES, _LANES)
                    idv = idsb[sl]
                    local = idv - ET
                    m = (local >= 0) & (local < TAIL)
                    plsc.addupdate_scatter(
                        tailb, [jnp.full((_LANES,), r * TAILP, jnp.int32)
                                + local], attnb[sl], mask=m)
            pltpu.sync_copy(tailb, tail_hbm.at[pl.ds(r0 * TAILP, _SUB * TAILP)])

            def fire_in(buf, sem, blk):
                def f(j, c):
                    pltpu.make_async_copy(
                        vocab_hbm.at[pl.ds(r0, _SUB),
                                     pl.ds((blk * TB + j) * _TILE, _TILE)],
                        buf.at[j], sem).start()
                    return c
                lax.fori_loop(0, nin_of(blk), f, None)

            def wait_in(buf, sem, blk):
                def f(j, c):
                    pltpu.make_async_copy(
                        vocab_hbm.at[pl.ds(r0, _SUB),
                                     pl.ds((blk * TB + j) * _TILE, _TILE)],
                        buf.at[j], sem).wait()
                    return c
                lax.fori_loop(0, nin_of(blk), f, None)

            def fire_out(buf, sem, blk):
                tb = jnp.minimum(TB, NT - blk * TB)

                def f(j, c):
                    pltpu.make_async_copy(
                        buf.at[j],
                        out_hbm.at[pl.ds(r0, _SUB),
                                   pl.ds((blk * TB + j) * _TILE, _TILE)],
                        sem).start()
                    return c
                lax.fori_loop(0, tb, f, None)

            def wait_out(buf, sem, blk):
                tb = jnp.minimum(TB, NT - blk * TB)

                def f(j, c):
                    pltpu.make_async_copy(
                        buf.at[j],
                        out_hbm.at[pl.ds(r0, _SUB),
                                   pl.ds((blk * TB + j) * _TILE, _TILE)],
                        sem).wait()
                    return c
                lax.fori_loop(0, tb, f, None)

            def process(buf, blk):
                tb = jnp.minimum(TB, NT - blk * TB)
                is_last = blk == NBLK - 1

                @pl.when(is_last)
                def _():
                    jsp = tb - 1
                    zero = jnp.zeros((_LANES,), jnp.float32)
                    for r in range(_SUB):
                        for c in range(VTW // _LANES):
                            buf[jsp, r, pl.ds(c * _LANES, _LANES)] = (
                                vtb[pl.ds(r * VTW + c * _LANES, _LANES)])
                        for c in range(VTW // _LANES, _TILE // _LANES):
                            buf[jsp, r, pl.ds(c * _LANES, _LANES)] = zero

                def scale(j, c3):
                    for r in range(_SUB):
                        for c in range(_TILE // _LANES):
                            sl = pl.ds(c * _LANES, _LANES)
                            buf[j, r, sl] = buf[j, r, sl] * pgs[r]
                    return c3
                lax.fori_loop(0, tb, scale, None)

                lo = blk * TB * _TILE
                hi = lo + tb * _TILE
                for r in range(_SUB):
                    rvec = jnp.full((_LANES,), r, jnp.int32)
                    for j in range(LP // _LANES):
                        sl = pl.ds(r * LP + j * _LANES, _LANES)
                        idv = idsb[sl]
                        local = idv - lo
                        m = (idv >= lo) & (idv < hi)
                        tv = lax.shift_right_logical(local, 7)
                        lv = lax.bitwise_and(local, 127)
                        plsc.addupdate_scatter(
                            buf, [tv, rvec, lv], attnb[sl], mask=m)

            fire_in(buf0, si0, 0)
            fire_in(buf1, si1, 1)

            def blk_step(g, c2):
                for b in range(3):
                    @pl.when(g % 3 == b)
                    def _():
                        wait_in(bufs[b], isems[b], g)
                        process(bufs[b], g)
                        fire_out(bufs[b], osems[b], g)
                        bp = (b + 2) % 3

                        @pl.when(g + 2 < NBLK)
                        def _():
                            @pl.when(g >= 1)
                            def _():
                                wait_out(bufs[bp], osems[bp], g - 1)
                            fire_in(bufs[bp], isems[bp], g + 2)
                return c2

            lax.fori_loop(0, NBLK, blk_step, None)
            for blk in (NBLK - 3, NBLK - 2, NBLK - 1):
                wait_out(bufs[blk % 3], osems[blk % 3], blk)
            return carry

        lax.fori_loop(0, SPW, stripe_body, None)

    return sc_fn, ET, TAIL, TAILP


def _tail_merge_tc(B, E, ET):
    jlast = ET // _TILE

    def body(t_ref, o_in_ref, o_ref):
        o_ref[...] = t_ref[...]

    return pl.pallas_call(
        body,
        grid=(B // _SUB,),
        in_specs=[
            pl.BlockSpec((_SUB, _TILE), lambda i: (i, 0)),
            pl.BlockSpec((_SUB, _TILE), lambda i: (i, jlast)),
        ],
        out_specs=pl.BlockSpec((_SUB, _TILE), lambda i: (i, jlast)),
        out_shape=jax.ShapeDtypeStruct((B, E), jnp.float32),
        input_output_aliases={1: 0},
    )


def kernel(vocab_dists, attn_dists, p_gens, input_ids):
    T, B, V = vocab_dists.shape
    L = attn_dists.shape[-1]
    E = V + _OOV
    LP = ((L + _LANES - 1) // _LANES) * _LANES

    sc_fn, ET, TAIL, TAILP = _final_dist_sc_tiled(B, V, L)
    merge = _tail_merge_tc(B, E, ET)
    NVT = V // _TILE

    outs = []
    for t in range(T):
        vocab = vocab_dists[t]
        vt = vocab[:, NVT * _TILE:].reshape(-1)
        pg_flat = jnp.broadcast_to(
            p_gens[t], (B, _LANES)).reshape(-1)
        attn_flat = jnp.pad(
            attn_dists[t], ((0, 0), (0, LP - L))).reshape(-1)
        ids_flat = jnp.pad(
            input_ids, ((0, 0), (0, LP - L)),
            constant_values=-1).reshape(-1)
        out1, tail = sc_fn(vocab, attn_flat, ids_flat, pg_flat, vt)
        tail4 = jnp.pad(tail.reshape(B, TAILP)[:, :TAIL],
                        ((0, 0), (0, _TILE - TAIL)))
        outs.append(merge(tail4, out1))
    return jnp.stack(outs, axis=0)
